# baseline (speedup 1.0000x reference)
import functools

import jax
import jax.numpy as jnp
from jax import lax
from jax.experimental import pallas as pl
from jax.experimental.pallas import tpu as pltpu
from jax.experimental.pallas import tpu_sc as plsc

E = 320000
V = 10000
D = 128
DW = D // 2
NC, NS = 2, 16
NW = NC * NS
EPW = E // NW
C = 200
PIECES = ((0, 104), (104, 96))
NCHUNK = EPW // C


def _dot_chunk(rows1_v, rows2_v, out_v, obase):
    lane15 = lax.iota(jnp.int32, 16) == 15

    def body(e, _):
        acc1 = jnp.zeros((16,), jnp.float32)
        acc2 = jnp.zeros((16,), jnp.float32)
        for w in range(D // 32):
            a = rows1_v[e, pl.ds(32 * w, 32)]
            b = rows2_v[e, pl.ds(32 * w, 32)]
            pa, pb = plsc.unpack(a * b, format=plsc.PackFormat.INTERLEAVED)
            acc1 = acc1 + pa
            acc2 = acc2 + pb
        csum = plsc.cumsum(acc1 + acc2)
        plsc.store_scatter(
            out_v, [jnp.full((16,), obase + e, jnp.int32)], csum, mask=lane15
        )
        return 0

    lax.fori_loop(0, C, body, 0, unroll=4)


def kernel(emb, edge_index):
    src = edge_index[0].astype(jnp.int32)
    dst = edge_index[1].astype(jnp.int32)
    emb_bf = emb.astype(jnp.bfloat16)

    mesh = plsc.VectorSubcoreMesh(core_axis_name="c", subcore_axis_name="s")

    @functools.partial(
        pl.kernel,
        mesh=mesh,
        out_type=jax.ShapeDtypeStruct((E,), jnp.float32),
        compiler_params=pltpu.CompilerParams(
            needs_layout_passes=False, use_tc_tiling_on_sc=False
        ),
        scratch_types=[
            pltpu.VMEM((EPW,), jnp.int32),
            pltpu.VMEM((EPW,), jnp.int32),
            pltpu.VMEM((C, D), jnp.bfloat16),
            pltpu.VMEM((C, D), jnp.bfloat16),
            pltpu.VMEM((C, D), jnp.bfloat16),
            pltpu.VMEM((C, D), jnp.bfloat16),
            pltpu.VMEM((EPW,), jnp.float32),
            pltpu.SemaphoreType.DMA,
            pltpu.SemaphoreType.DMA,
            pltpu.SemaphoreType.DMA,
            pltpu.SemaphoreType.DMA,
        ],
    )
    def _k(emb_hbm, src_hbm, dst_hbm, out_hbm,
           idx1_all, idx2_all, r1a, r1b, r2a, r2b, out_all,
           s1a, s1b, s2a, s2b):
        wid = lax.axis_index("s") * NC + lax.axis_index("c")
        wbase = wid * EPW

        pltpu.sync_copy(src_hbm.at[pl.ds(wbase, EPW)], idx1_all)
        pltpu.sync_copy(dst_hbm.at[pl.ds(wbase, EPW)], idx2_all)

        def fire(i, r1, r2, s1, s2):
            for po, pn in PIECES:
                off = pl.ds(i * C + po, pn)
                dst = pl.ds(po, pn)
                pltpu.async_copy(emb_hbm.at[idx1_all.at[off]], r1.at[dst], s1)
                pltpu.async_copy(emb_hbm.at[idx2_all.at[off]], r2.at[dst], s2)

        def wait(r1, r2, s1, s2):
            for po, pn in PIECES:
                off = pl.ds(po, pn)
                dst = pl.ds(po, pn)
                pltpu.make_async_copy(
                    emb_hbm.at[idx1_all.at[off]], r1.at[dst], s1).wait()
                pltpu.make_async_copy(
                    emb_hbm.at[idx2_all.at[off]], r2.at[dst], s2).wait()

        fire(0, r1a, r2a, s1a, s2a)

        def body(k, _):
            i0 = 2 * k
            with jax.named_scope("wait_a"):
                wait(r1a, r2a, s1a, s2a)
            with jax.named_scope("fire_b"):
                fire(i0 + 1, r1b, r2b, s1b, s2b)
            with jax.named_scope("dot_a"):
                _dot_chunk(r1a, r2a, out_all, i0 * C)
            with jax.named_scope("wait_b"):
                wait(r1b, r2b, s1b, s2b)

            @pl.when(i0 + 2 < NCHUNK)
            def _():
                fire(i0 + 2, r1a, r2a, s1a, s2a)

            with jax.named_scope("dot_b"):
                _dot_chunk(r1b, r2b, out_all, (i0 + 1) * C)
            return 0

        lax.fori_loop(0, NCHUNK // 2, body, 0)

        pltpu.sync_copy(out_all, out_hbm.at[pl.ds(wbase, EPW)])

    return _k(emb_bf, src, dst)


# speedup vs baseline: 1.3715x; 1.3715x over previous
import functools

import jax
import jax.numpy as jnp
from jax import lax
from jax.experimental import pallas as pl
from jax.experimental.pallas import tpu as pltpu
from jax.experimental.pallas import tpu_sc as plsc

E = 320000
V = 10000
D = 128
DW = D // 2
NC, NS = 2, 16
NW = NC * NS
EPW = E // NW
C = 80
G = C // 16
NCHUNK = EPW // C


def _dot_chunk(rows1_v, rows2_v, out_v, obase):
    lanes = lax.iota(jnp.int32, 16)

    def group(g, _):
        e0 = g * 16
        outv = jnp.zeros((16,), jnp.float32)
        for j in range(16):
            e = e0 + j
            acc1 = jnp.zeros((16,), jnp.float32)
            acc2 = jnp.zeros((16,), jnp.float32)
            for w in range(D // 32):
                a = rows1_v[e, pl.ds(32 * w, 32)]
                b = rows2_v[e, pl.ds(32 * w, 32)]
                pa, pb = plsc.unpack(a * b, format=plsc.PackFormat.INTERLEAVED)
                acc1 = acc1 + pa
                acc2 = acc2 + pb
            outv = jnp.where(lanes == j, jnp.sum(acc1 + acc2), outv)
        plsc.store_scatter(out_v, [obase + e0 + lanes], outv)
        return 0

    lax.fori_loop(0, G, group, 0)


def kernel(emb, edge_index):
    src = edge_index[0].astype(jnp.int32)
    dst = edge_index[1].astype(jnp.int32)
    emb_bf = emb.astype(jnp.bfloat16)

    mesh = plsc.VectorSubcoreMesh(core_axis_name="c", subcore_axis_name="s")

    @functools.partial(
        pl.kernel,
        mesh=mesh,
        out_type=jax.ShapeDtypeStruct((E,), jnp.float32),
        compiler_params=pltpu.CompilerParams(
            needs_layout_passes=False, use_tc_tiling_on_sc=False
        ),
        scratch_types=[
            pltpu.VMEM((EPW,), jnp.int32),
            pltpu.VMEM((EPW,), jnp.int32),
            pltpu.VMEM((C, D), jnp.bfloat16),
            pltpu.VMEM((C, D), jnp.bfloat16),
            pltpu.VMEM((C, D), jnp.bfloat16),
            pltpu.VMEM((C, D), jnp.bfloat16),
            pltpu.VMEM((EPW,), jnp.float32),
            pltpu.SemaphoreType.DMA,
            pltpu.SemaphoreType.DMA,
            pltpu.SemaphoreType.DMA,
            pltpu.SemaphoreType.DMA,
        ],
    )
    def _k(emb_hbm, src_hbm, dst_hbm, out_hbm,
           idx1_all, idx2_all, r1a, r1b, r2a, r2b, out_all,
           s1a, s1b, s2a, s2b):
        wid = lax.axis_index("s") * NC + lax.axis_index("c")
        wbase = wid * EPW

        pltpu.sync_copy(src_hbm.at[pl.ds(wbase, EPW)], idx1_all)
        pltpu.sync_copy(dst_hbm.at[pl.ds(wbase, EPW)], idx2_all)

        def fire(i, r1, r2, s1, s2):
            off = pl.ds(i * C, C)
            pltpu.async_copy(emb_hbm.at[idx1_all.at[off]], r1, s1)
            pltpu.async_copy(emb_hbm.at[idx2_all.at[off]], r2, s2)

        def wait(r1, r2, s1, s2):
            off = pl.ds(0, C)
            pltpu.make_async_copy(emb_hbm.at[idx1_all.at[off]], r1, s1).wait()
            pltpu.make_async_copy(emb_hbm.at[idx2_all.at[off]], r2, s2).wait()

        fire(0, r1a, r2a, s1a, s2a)

        def body(k, _):
            i0 = 2 * k
            wait(r1a, r2a, s1a, s2a)
            fire(i0 + 1, r1b, r2b, s1b, s2b)
            _dot_chunk(r1a, r2a, out_all, i0 * C)
            wait(r1b, r2b, s1b, s2b)
            fire(i0 + 2, r1a, r2a, s1a, s2a)
            _dot_chunk(r1b, r2b, out_all, (i0 + 1) * C)
            return 0

        lax.fori_loop(0, (NCHUNK - 1) // 2, body, 0)

        wait(r1a, r2a, s1a, s2a)
        _dot_chunk(r1a, r2a, out_all, (NCHUNK - 1) * C)

        pltpu.sync_copy(out_all, out_hbm.at[pl.ds(wbase, EPW)])

    return _k(emb_bf, src, dst)


# speedup vs baseline: 1.4508x; 1.0578x over previous
import functools

import jax
import jax.numpy as jnp
from jax import lax
from jax.experimental import pallas as pl
from jax.experimental.pallas import tpu as pltpu
from jax.experimental.pallas import tpu_sc as plsc

E = 320000
V = 10000
D = 128
NC, NS = 2, 16
NW = NC * NS
EPW = E // NW
C = 80
G = C // 16
NCHUNK = EPW // C


def _dot_chunk(rows1_v, rows2_v, out_v, obase):
    lanes = lax.iota(jnp.int32, 16)

    def group(g, _):
        e0 = g * 16
        outv = jnp.zeros((16,), jnp.float32)
        for j in range(16):
            e = e0 + j
            p = []
            for w in range(D // 32):
                a = rows1_v[e, pl.ds(32 * w, 32)]
                b = rows2_v[e, pl.ds(32 * w, 32)]
                p.append(a * b)
            ps = (p[0] + p[1]) + (p[2] + p[3])
            pa, pb = plsc.unpack(ps, format=plsc.PackFormat.INTERLEAVED)
            outv = jnp.where(lanes == j, jnp.sum(pa + pb), outv)
        plsc.store_scatter(out_v, [obase + e0 + lanes], outv)
        return 0

    lax.fori_loop(0, G, group, 0)


def kernel(emb, edge_index):
    emb_bf = emb.astype(jnp.bfloat16)
    ei = edge_index.astype(jnp.int32)

    mesh = plsc.VectorSubcoreMesh(core_axis_name="c", subcore_axis_name="s")

    @functools.partial(
        pl.kernel,
        mesh=mesh,
        out_type=jax.ShapeDtypeStruct((E,), jnp.float32),
        compiler_params=pltpu.CompilerParams(
            needs_layout_passes=False, use_tc_tiling_on_sc=False
        ),
        scratch_types=[
            pltpu.VMEM((EPW,), jnp.int32),
            pltpu.VMEM((EPW,), jnp.int32),
            pltpu.VMEM((C, D), jnp.bfloat16),
            pltpu.VMEM((C, D), jnp.bfloat16),
            pltpu.VMEM((C, D), jnp.bfloat16),
            pltpu.VMEM((C, D), jnp.bfloat16),
            pltpu.VMEM((EPW,), jnp.float32),
            pltpu.SemaphoreType.DMA,
            pltpu.SemaphoreType.DMA,
            pltpu.SemaphoreType.DMA,
            pltpu.SemaphoreType.DMA,
        ],
    )
    def _k(emb_hbm, ei_hbm, out_hbm,
           idx1_all, idx2_all, r1a, r1b, r2a, r2b, out_all,
           s1a, s1b, s2a, s2b):
        wid = lax.axis_index("s") * NC + lax.axis_index("c")
        wbase = wid * EPW

        pltpu.sync_copy(ei_hbm.at[0, pl.ds(wbase, EPW)], idx1_all)
        pltpu.sync_copy(ei_hbm.at[1, pl.ds(wbase, EPW)], idx2_all)

        def fire(i, r1, r2, s1, s2):
            off = pl.ds(i * C, C)
            pltpu.async_copy(emb_hbm.at[idx1_all.at[off]], r1, s1)
            pltpu.async_copy(emb_hbm.at[idx2_all.at[off]], r2, s2)

        def wait(r1, r2, s1, s2):
            off = pl.ds(0, C)
            pltpu.make_async_copy(emb_hbm.at[idx1_all.at[off]], r1, s1).wait()
            pltpu.make_async_copy(emb_hbm.at[idx2_all.at[off]], r2, s2).wait()

        fire(0, r1a, r2a, s1a, s2a)

        def body(k, _):
            i0 = 2 * k
            wait(r1a, r2a, s1a, s2a)
            fire(i0 + 1, r1b, r2b, s1b, s2b)
            _dot_chunk(r1a, r2a, out_all, i0 * C)
            wait(r1b, r2b, s1b, s2b)
            fire(i0 + 2, r1a, r2a, s1a, s2a)
            _dot_chunk(r1b, r2b, out_all, (i0 + 1) * C)
            return 0

        lax.fori_loop(0, (NCHUNK - 1) // 2, body, 0)

        wait(r1a, r2a, s1a, s2a)
        _dot_chunk(r1a, r2a, out_all, (NCHUNK - 1) * C)

        pltpu.sync_copy(out_all, out_hbm.at[pl.ds(wbase, EPW)])

    return _k(emb_bf, ei)


# speedup vs baseline: 1.4551x; 1.0030x over previous
import functools

import jax
import jax.numpy as jnp
from jax import lax
from jax.experimental import pallas as pl
from jax.experimental.pallas import tpu as pltpu
from jax.experimental.pallas import tpu_sc as plsc

E = 320000
V = 10000
D = 128
NC, NS = 2, 16
NW = NC * NS
EPW = E // NW
C = 80
G = C // 16
NCHUNK = EPW // C


def _tree_order():
    vecs = [[s] * 16 for s in range(16)]
    for w in (8, 4, 2, 1):
        vecs = [
            [x[l] if (l // w) % 2 == 0 else y[l] for l in range(16)]
            for x, y in zip(vecs[0::2], vecs[1::2])
        ]
    sigma = vecs[0]
    order = [0] * 16
    for l in range(16):
        order[sigma[l]] = l
    return order


_ORDER = _tree_order()


def _dot_chunk(rows1_v, rows2_v, out_v, obase):
    lanes = lax.iota(jnp.int32, 16)
    xor_idx = {w: lanes ^ w for w in (8, 4, 2, 1)}
    sel_mask = {w: (lanes // w) % 2 == 0 for w in (8, 4, 2, 1)}

    def group(g, _):
        e0 = g * 16
        vs = []
        for j in _ORDER:
            e = e0 + j
            p = []
            for w in range(D // 32):
                a = rows1_v[e, pl.ds(32 * w, 32)]
                b = rows2_v[e, pl.ds(32 * w, 32)]
                p.append(a * b)
            ps = (p[0] + p[1]) + (p[2] + p[3])
            pa, pb = plsc.unpack(ps, format=plsc.PackFormat.INTERLEAVED)
            vs.append(pa + pb)
        for w in (8, 4, 2, 1):
            vs = [
                jnp.where(
                    sel_mask[w],
                    x + x.at[xor_idx[w]].get(mode="promise_in_bounds"),
                    y + y.at[xor_idx[w]].get(mode="promise_in_bounds"),
                )
                for x, y in zip(vs[0::2], vs[1::2])
            ]
        plsc.store_scatter(out_v, [obase + e0 + lanes], vs[0])
        return 0

    lax.fori_loop(0, G, group, 0)


def kernel(emb, edge_index):
    emb_bf = emb.astype(jnp.bfloat16)
    ei = edge_index.astype(jnp.int32)

    mesh = plsc.VectorSubcoreMesh(core_axis_name="c", subcore_axis_name="s")

    @functools.partial(
        pl.kernel,
        mesh=mesh,
        out_type=jax.ShapeDtypeStruct((E,), jnp.float32),
        compiler_params=pltpu.CompilerParams(
            needs_layout_passes=False, use_tc_tiling_on_sc=False
        ),
        scratch_types=[
            pltpu.VMEM((EPW,), jnp.int32),
            pltpu.VMEM((EPW,), jnp.int32),
            pltpu.VMEM((C, D), jnp.bfloat16),
            pltpu.VMEM((C, D), jnp.bfloat16),
            pltpu.VMEM((C, D), jnp.bfloat16),
            pltpu.VMEM((C, D), jnp.bfloat16),
            pltpu.VMEM((EPW,), jnp.float32),
            pltpu.SemaphoreType.DMA,
            pltpu.SemaphoreType.DMA,
            pltpu.SemaphoreType.DMA,
            pltpu.SemaphoreType.DMA,
        ],
    )
    def _k(emb_hbm, ei_hbm, out_hbm,
           idx1_all, idx2_all, r1a, r1b, r2a, r2b, out_all,
           s1a, s1b, s2a, s2b):
        wid = lax.axis_index("s") * NC + lax.axis_index("c")
        wbase = wid * EPW

        pltpu.sync_copy(ei_hbm.at[0, pl.ds(wbase, EPW)], idx1_all)
        pltpu.sync_copy(ei_hbm.at[1, pl.ds(wbase, EPW)], idx2_all)

        def fire(i, r1, r2, s1, s2):
            off = pl.ds(i * C, C)
            pltpu.async_copy(emb_hbm.at[idx1_all.at[off]], r1, s1)
            pltpu.async_copy(emb_hbm.at[idx2_all.at[off]], r2, s2)

        def wait(r1, r2, s1, s2):
            off = pl.ds(0, C)
            pltpu.make_async_copy(emb_hbm.at[idx1_all.at[off]], r1, s1).wait()
            pltpu.make_async_copy(emb_hbm.at[idx2_all.at[off]], r2, s2).wait()

        fire(0, r1a, r2a, s1a, s2a)

        def body(k, _):
            i0 = 2 * k
            wait(r1a, r2a, s1a, s2a)
            fire(i0 + 1, r1b, r2b, s1b, s2b)
            _dot_chunk(r1a, r2a, out_all, i0 * C)
            wait(r1b, r2b, s1b, s2b)
            fire(i0 + 2, r1a, r2a, s1a, s2a)
            _dot_chunk(r1b, r2b, out_all, (i0 + 1) * C)
            return 0

        lax.fori_loop(0, (NCHUNK - 1) // 2, body, 0)

        wait(r1a, r2a, s1a, s2a)
        _dot_chunk(r1a, r2a, out_all, (NCHUNK - 1) * C)

        pltpu.sync_copy(out_all, out_hbm.at[pl.ds(wbase, EPW)])

    return _k(emb_bf, ei)


# speedup vs baseline: 1.4562x; 1.0008x over previous
import functools

import jax
import jax.numpy as jnp
from jax import lax
from jax.experimental import pallas as pl
from jax.experimental.pallas import tpu as pltpu
from jax.experimental.pallas import tpu_sc as plsc

E = 320000
V = 10000
D = 128
NC, NS = 2, 16
NW = NC * NS
EPW = E // NW
C = 80
G = C // 16
NCHUNK = EPW // C
NBUF = 5


def _tree_order():
    vecs = [[s] * 16 for s in range(16)]
    for w in (8, 4, 2, 1):
        vecs = [
            [x[l] if (l // w) % 2 == 0 else y[l] for l in range(16)]
            for x, y in zip(vecs[0::2], vecs[1::2])
        ]
    sigma = vecs[0]
    order = [0] * 16
    for l in range(16):
        order[sigma[l]] = l
    return order


_ORDER = _tree_order()


def _dot_chunk(rows1_v, rows2_v, out_v, obase):
    lanes = lax.iota(jnp.int32, 16)
    xor_idx = {w: lanes ^ w for w in (8, 4, 2, 1)}
    sel_mask = {w: (lanes // w) % 2 == 0 for w in (8, 4, 2, 1)}

    def group(g, _):
        e0 = g * 16
        vs = []
        for j in _ORDER:
            e = e0 + j
            p = []
            for w in range(D // 32):
                a = rows1_v[e, pl.ds(32 * w, 32)]
                b = rows2_v[e, pl.ds(32 * w, 32)]
                p.append(a * b)
            ps = (p[0] + p[1]) + (p[2] + p[3])
            pa, pb = plsc.unpack(ps, format=plsc.PackFormat.INTERLEAVED)
            vs.append(pa + pb)
        for w in (8, 4, 2, 1):
            vs = [
                jnp.where(
                    sel_mask[w],
                    x + x.at[xor_idx[w]].get(mode="promise_in_bounds"),
                    y + y.at[xor_idx[w]].get(mode="promise_in_bounds"),
                )
                for x, y in zip(vs[0::2], vs[1::2])
            ]
        plsc.store_scatter(out_v, [obase + e0 + lanes], vs[0])
        return 0

    lax.fori_loop(0, G, group, 0)


def kernel(emb, edge_index):
    emb_bf = emb.astype(jnp.bfloat16)
    ei = edge_index.astype(jnp.int32)

    mesh = plsc.VectorSubcoreMesh(core_axis_name="c", subcore_axis_name="s")

    @functools.partial(
        pl.kernel,
        mesh=mesh,
        out_type=jax.ShapeDtypeStruct((E,), jnp.float32),
        compiler_params=pltpu.CompilerParams(
            needs_layout_passes=False, use_tc_tiling_on_sc=False
        ),
        scratch_types=[
            pltpu.VMEM((EPW,), jnp.int32),
            pltpu.VMEM((EPW,), jnp.int32),
            [pltpu.VMEM((C, D), jnp.bfloat16)] * NBUF,
            [pltpu.VMEM((C, D), jnp.bfloat16)] * NBUF,
            pltpu.VMEM((EPW,), jnp.float32),
            [pltpu.SemaphoreType.DMA] * NBUF,
            [pltpu.SemaphoreType.DMA] * NBUF,
        ],
    )
    def _k(emb_hbm, ei_hbm, out_hbm,
           idx1_all, idx2_all, r1s, r2s, out_all, s1s, s2s):
        wid = lax.axis_index("s") * NC + lax.axis_index("c")
        wbase = wid * EPW

        pltpu.sync_copy(ei_hbm.at[0, pl.ds(wbase, EPW)], idx1_all)
        pltpu.sync_copy(ei_hbm.at[1, pl.ds(wbase, EPW)], idx2_all)

        def fire(i, r):
            off = pl.ds(i * C, C)
            pltpu.async_copy(emb_hbm.at[idx1_all.at[off]], r1s[r], s1s[r])
            pltpu.async_copy(emb_hbm.at[idx2_all.at[off]], r2s[r], s2s[r])

        def wait(r):
            off = pl.ds(0, C)
            pltpu.make_async_copy(
                emb_hbm.at[idx1_all.at[off]], r1s[r], s1s[r]).wait()
            pltpu.make_async_copy(
                emb_hbm.at[idx2_all.at[off]], r2s[r], s2s[r]).wait()

        for r in range(NBUF - 1):
            fire(r, r)

        def body(k, _):
            i0 = NBUF * k
            for r in range(NBUF):
                i = i0 + r
                wait(r)

                @pl.when(i + NBUF - 1 < NCHUNK)
                def _():
                    fire(i + NBUF - 1, (r + NBUF - 1) % NBUF)

                _dot_chunk(r1s[r], r2s[r], out_all, i * C)
            return 0

        lax.fori_loop(0, NCHUNK // NBUF, body, 0)

        pltpu.sync_copy(out_all, out_hbm.at[pl.ds(wbase, EPW)])

    return _k(emb_bf, ei)
